# Initial kernel scaffold; baseline (speedup 1.0000x reference)
import jax
import jax.numpy as jnp
from jax.experimental import pallas as pl


def kernel(input_pc, params):
    raise NotImplementedError("write your pallas kernel here")


# speedup vs baseline: 1.1100x; 1.1100x over previous
import functools

import jax
import jax.numpy as jnp
from jax.experimental import pallas as pl
from jax.experimental.pallas import tpu as pltpu

_SA_CFG = [
    (1024, (0.1, 0.2), (16, 32), ((3, 16, 16, 32), (3, 16, 16, 32))),
    (768, (0.2, 0.4), (32, 64), ((67, 32, 32, 64), (67, 32, 32, 64))),
    (512, (0.4, 0.6), (32, 64), ((131, 64, 64, 128), (131, 64, 64, 128))),
    (512, (0.6, 0.8), (64, 128), ((259, 128, 128, 256), (259, 128, 128, 256))),
]


def _fold(p):
    s = p["g"] * jax.lax.rsqrt(p["v"] + 1e-5)
    w = p["W"] * s[:, None]
    b = (p["b"] - p["m"]) * s + p["be"]
    return w.T, b[None, :]


def _head_body(ctx_ref, xyzt_ref, *refs):
    wb, outs = refs[:10], refs[10:]
    skel_ref, r_ref, cmb_ref = outs
    h = ctx_ref[...]
    for i in range(5):
        w, b = wb[2 * i][...], wb[2 * i + 1][...]
        h = jnp.dot(h, w, preferred_element_type=jnp.float32) + b
        if i < 4:
            h = jnp.maximum(h, 0.0)
    m = jnp.max(h, axis=0, keepdims=True)
    e = jnp.exp(h - m)
    sm = e / jnp.sum(e, axis=0, keepdims=True)
    xyzt = xyzt_ref[...]
    skel = jax.lax.dot_general(sm, xyzt, (((0,), (1,)), ((), ())),
                               preferred_element_type=jnp.float32)
    cmb = jax.lax.dot_general(sm, ctx_ref[...], (((0,), (0,)), ((), ())),
                              preferred_element_type=jnp.float32)
    dx = skel[:, 0:1] - xyzt[0:1, :]
    dy = skel[:, 1:2] - xyzt[1:2, :]
    dz = skel[:, 2:3] - xyzt[2:3, :]
    dt = jnp.sqrt(dx * dx + dy * dy + dz * dz + 1e-12)
    mind = jnp.min(dt, axis=0, keepdims=True)
    skel_r = jax.lax.dot_general(sm, mind, (((0,), (1,)), ((), ())),
                                 preferred_element_type=jnp.float32)
    skel_ref[...] = skel
    r_ref[...] = skel_r
    cmb_ref[...] = cmb


def _head(context, xyzt, head_params):
    b, s, c = context.shape
    k = head_params[-1]["W"].shape[0]
    wbs = []
    for p in head_params:
        w, bb = _fold(p)
        wbs += [w, bb]
    spec2 = lambda shp: pl.BlockSpec(shp, lambda i: (i,) + (0,) * len(shp))
    grid_spec = pl.GridSpec(
        grid=(b,),
        in_specs=[
            pl.BlockSpec((1, s, c), lambda i: (i, 0, 0)),
            pl.BlockSpec((1, 3, s), lambda i: (i, 0, 0)),
        ] + [pl.BlockSpec(w.shape, lambda i: (0,) * w.ndim) for w in wbs],
        out_specs=[
            pl.BlockSpec((1, k, 3), lambda i: (i, 0, 0)),
            pl.BlockSpec((1, k, 1), lambda i: (i, 0, 0)),
            pl.BlockSpec((1, k, c), lambda i: (i, 0, 0)),
        ],
    )
    del spec2

    def body(ctx_ref, xyzt_ref, *refs):
        _head_body(ctx_ref.at[0], xyzt_ref.at[0], *refs[:10],
                   refs[10].at[0], refs[11].at[0], refs[12].at[0])

    out = pl.pallas_call(
        body,
        grid_spec=grid_spec,
        out_shape=[
            jax.ShapeDtypeStruct((b, k, 3), jnp.float32),
            jax.ShapeDtypeStruct((b, k, 1), jnp.float32),
            jax.ShapeDtypeStruct((b, k, c), jnp.float32),
        ],
    )(context, xyzt, *wbs)
    return out


def _fps0(xyz, npoint):
    def single(x):
        def body(i, carry):
            idxs, dists, last = carry
            d = jnp.sum((x - x[last]) ** 2, axis=1)
            dists = jnp.minimum(dists, d)
            nxt = jnp.argmax(dists).astype(jnp.int32)
            return idxs.at[i].set(nxt), dists, nxt

        idxs = jnp.zeros((npoint,), jnp.int32)
        dists = jnp.full((x.shape[0],), 1e10, jnp.float32)
        idxs, _, _ = jax.lax.fori_loop(1, npoint, body, (idxs, dists, jnp.int32(0)))
        return idxs

    return jax.vmap(single)(xyz)


def _ball0(radius, nsample, xyz, new_xyz):
    sqr = jnp.sum((new_xyz[:, :, None, :] - xyz[:, None, :, :]) ** 2, axis=-1)
    mask = sqr < radius * radius
    order = jnp.argsort((~mask).astype(jnp.int32), axis=-1)[:, :, :nsample]
    cnt = jnp.sum(mask, axis=-1, keepdims=True)
    valid = jnp.arange(nsample)[None, None, :] < cnt
    return jnp.where(valid, order, order[:, :, :1])


def _take0(x, idx):
    return jax.vmap(lambda a, i: a[i])(x, idx)


def _bn0(x, p, relu=True):
    y = (x - p["m"]) / jnp.sqrt(p["v"] + 1e-5) * p["g"] + p["be"]
    return jnp.maximum(y, 0.0) if relu else y


def _sa0(xyz, feats, cfg, scale_params):
    npoint, radii, nsamples, _ = cfg
    fps_idx = _fps0(xyz, npoint)
    new_xyz = _take0(xyz, fps_idx)
    outs = []
    for radius, nsample, layers in zip(radii, nsamples, scale_params):
        idx = _ball0(radius, nsample, xyz, new_xyz)
        grouped = _take0(xyz, idx) - new_xyz[:, :, None, :]
        if feats is not None:
            grouped = jnp.concatenate([grouped, _take0(feats, idx)], axis=-1)
        h = grouped
        for lp in layers:
            h = _bn0(h @ lp["W"].T + lp["b"], lp)
        outs.append(jnp.max(h, axis=2))
    return new_xyz, jnp.concatenate(outs, axis=-1)


def kernel(input_pc, params):
    xyz = input_pc[..., 0:3]
    feats = None
    for cfg, sp in zip(_SA_CFG, params["sa"]):
        xyz, feats = _sa0(xyz, feats, cfg, sp)
    xyzt = jnp.transpose(xyz, (0, 2, 1))
    skel_xyz, skel_r, shape_cmb = _head(feats, xyzt, params["head"])
    return skel_xyz, skel_r, shape_cmb


# speedup vs baseline: 1.4444x; 1.3013x over previous
import functools

import jax
import jax.numpy as jnp
from jax.experimental import pallas as pl
from jax.experimental.pallas import tpu as pltpu

_SA_CFG = [
    (1024, (0.1, 0.2), (16, 32), ((3, 16, 16, 32), (3, 16, 16, 32))),
    (768, (0.2, 0.4), (32, 64), ((67, 32, 32, 64), (67, 32, 32, 64))),
    (512, (0.4, 0.6), (32, 64), ((131, 64, 64, 128), (131, 64, 64, 128))),
    (512, (0.6, 0.8), (64, 128), ((259, 128, 128, 256), (259, 128, 128, 256))),
]


def _fold(p):
    s = p["g"] * jax.lax.rsqrt(p["v"] + 1e-5)
    w = p["W"] * s[:, None]
    b = (p["b"] - p["m"]) * s + p["be"]
    return w.T, b[None, :]


def _head_body(ctx_ref, xyzt_ref, *refs):
    wb, outs = refs[:10], refs[10:]
    skel_ref, r_ref, cmb_ref = outs
    h = ctx_ref[...]
    for i in range(5):
        w, b = wb[2 * i][...], wb[2 * i + 1][...]
        h = jnp.dot(h, w, preferred_element_type=jnp.float32) + b
        if i < 4:
            h = jnp.maximum(h, 0.0)
    m = jnp.max(h, axis=0, keepdims=True)
    e = jnp.exp(h - m)
    sm = e / jnp.sum(e, axis=0, keepdims=True)
    xyzt = xyzt_ref[...]
    skel = jax.lax.dot_general(sm, xyzt, (((0,), (1,)), ((), ())),
                               preferred_element_type=jnp.float32)
    cmb = jax.lax.dot_general(sm, ctx_ref[...], (((0,), (0,)), ((), ())),
                              preferred_element_type=jnp.float32)
    dx = skel[:, 0:1] - xyzt[0:1, :]
    dy = skel[:, 1:2] - xyzt[1:2, :]
    dz = skel[:, 2:3] - xyzt[2:3, :]
    dt = jnp.sqrt(dx * dx + dy * dy + dz * dz + 1e-12)
    mind = jnp.min(dt, axis=0, keepdims=True)
    skel_r = jax.lax.dot_general(sm, mind, (((0,), (1,)), ((), ())),
                                 preferred_element_type=jnp.float32)
    skel_ref[...] = skel
    r_ref[...] = skel_r
    cmb_ref[...] = cmb


def _head(context, xyzt, head_params):
    b, s, c = context.shape
    k = head_params[-1]["W"].shape[0]
    wbs = []
    for p in head_params:
        w, bb = _fold(p)
        wbs += [w, bb]
    spec2 = lambda shp: pl.BlockSpec(shp, lambda i: (i,) + (0,) * len(shp))
    grid_spec = pl.GridSpec(
        grid=(b,),
        in_specs=[
            pl.BlockSpec((1, s, c), lambda i: (i, 0, 0)),
            pl.BlockSpec((1, 3, s), lambda i: (i, 0, 0)),
        ] + [pl.BlockSpec(w.shape, lambda i: (0,) * w.ndim) for w in wbs],
        out_specs=[
            pl.BlockSpec((1, k, 3), lambda i: (i, 0, 0)),
            pl.BlockSpec((1, k, 1), lambda i: (i, 0, 0)),
            pl.BlockSpec((1, k, c), lambda i: (i, 0, 0)),
        ],
    )
    del spec2

    def body(ctx_ref, xyzt_ref, *refs):
        _head_body(ctx_ref.at[0], xyzt_ref.at[0], *refs[:10],
                   refs[10].at[0], refs[11].at[0], refs[12].at[0])

    out = pl.pallas_call(
        body,
        grid_spec=grid_spec,
        out_shape=[
            jax.ShapeDtypeStruct((b, k, 3), jnp.float32),
            jax.ShapeDtypeStruct((b, k, 1), jnp.float32),
            jax.ShapeDtypeStruct((b, k, c), jnp.float32),
        ],
    )(context, xyzt, *wbs)
    return out


def _fps_body(x_ref, o_ref, *, P, N8):
    x = x_ref[0]
    niota = (jax.lax.broadcasted_iota(jnp.int32, (8, N8), 0) * N8
             + jax.lax.broadcasted_iota(jnp.int32, (8, N8), 1))
    piota = jax.lax.broadcasted_iota(jnp.int32, (1, P), 1)
    col0 = x[:, 0:1, 0:1]
    acc0 = jnp.where(piota == 0, col0.reshape(3, 1), 0.0)

    def step(i, carry):
        dists, last, acc = carry
        diff = x - last
        d = (diff[0] * diff[0] + diff[1] * diff[1]) + diff[2] * diff[2]
        dists = jnp.minimum(dists, d)
        m = jnp.max(dists)
        nxt = jnp.min(jnp.where(dists == m, niota, jnp.int32(8 * N8)))
        sel = (niota == nxt)[None]
        col = jnp.sum(jnp.where(sel, x, 0.0), axis=(1, 2), keepdims=True)
        acc = jnp.where(piota == i, col.reshape(3, 1), acc)
        return dists, col, acc

    dists0 = jnp.full((8, N8), 1e10, jnp.float32)
    _, _, acc = jax.lax.fori_loop(1, P, step, (dists0, col0, acc0))
    o_ref[0] = acc


def _fps_t(xyzt, npoint):
    b, _, n = xyzt.shape
    n8 = n // 8
    xr = xyzt.reshape(b, 3, 8, n8)
    return pl.pallas_call(
        functools.partial(_fps_body, P=npoint, N8=n8),
        grid=(b,),
        in_specs=[pl.BlockSpec((1, 3, 8, n8), lambda i: (i, 0, 0, 0))],
        out_specs=pl.BlockSpec((1, 3, npoint), lambda i: (i, 0, 0)),
        out_shape=jax.ShapeDtypeStruct((b, 3, npoint), jnp.float32),
    )(xr)


def _fps0(xyz, npoint):
    def single(x):
        def body(i, carry):
            idxs, dists, last = carry
            d = jnp.sum((x - x[last]) ** 2, axis=1)
            dists = jnp.minimum(dists, d)
            nxt = jnp.argmax(dists).astype(jnp.int32)
            return idxs.at[i].set(nxt), dists, nxt

        idxs = jnp.zeros((npoint,), jnp.int32)
        dists = jnp.full((x.shape[0],), 1e10, jnp.float32)
        idxs, _, _ = jax.lax.fori_loop(1, npoint, body, (idxs, dists, jnp.int32(0)))
        return idxs

    return jax.vmap(single)(xyz)


def _ball0(radius, nsample, xyz, new_xyz):
    sqr = jnp.sum((new_xyz[:, :, None, :] - xyz[:, None, :, :]) ** 2, axis=-1)
    mask = sqr < radius * radius
    order = jnp.argsort((~mask).astype(jnp.int32), axis=-1)[:, :, :nsample]
    cnt = jnp.sum(mask, axis=-1, keepdims=True)
    valid = jnp.arange(nsample)[None, None, :] < cnt
    return jnp.where(valid, order, order[:, :, :1])


def _take0(x, idx):
    return jax.vmap(lambda a, i: a[i])(x, idx)


def _bn0(x, p, relu=True):
    y = (x - p["m"]) / jnp.sqrt(p["v"] + 1e-5) * p["g"] + p["be"]
    return jnp.maximum(y, 0.0) if relu else y


def _sa0(xyz, feats, cfg, scale_params):
    npoint, radii, nsamples, _ = cfg
    new_xyz = jnp.transpose(_fps_t(jnp.transpose(xyz, (0, 2, 1)), npoint),
                            (0, 2, 1))
    outs = []
    for radius, nsample, layers in zip(radii, nsamples, scale_params):
        idx = _ball0(radius, nsample, xyz, new_xyz)
        grouped = _take0(xyz, idx) - new_xyz[:, :, None, :]
        if feats is not None:
            grouped = jnp.concatenate([grouped, _take0(feats, idx)], axis=-1)
        h = grouped
        for lp in layers:
            h = _bn0(h @ lp["W"].T + lp["b"], lp)
        outs.append(jnp.max(h, axis=2))
    return new_xyz, jnp.concatenate(outs, axis=-1)


def kernel(input_pc, params):
    xyz = input_pc[..., 0:3]
    feats = None
    for cfg, sp in zip(_SA_CFG, params["sa"]):
        xyz, feats = _sa0(xyz, feats, cfg, sp)
    xyzt = jnp.transpose(xyz, (0, 2, 1))
    skel_xyz, skel_r, shape_cmb = _head(feats, xyzt, params["head"])
    return skel_xyz, skel_r, shape_cmb


# speedup vs baseline: 7.4761x; 5.1758x over previous
import functools

import jax
import jax.numpy as jnp
from jax.experimental import pallas as pl
from jax.experimental.pallas import tpu as pltpu
from jax.experimental.pallas import tpu_sc as plsc

_SA_CFG = [
    (1024, (0.1, 0.2), (16, 32), ((3, 16, 16, 32), (3, 16, 16, 32))),
    (768, (0.2, 0.4), (32, 64), ((67, 32, 32, 64), (67, 32, 32, 64))),
    (512, (0.4, 0.6), (32, 64), ((131, 64, 64, 128), (131, 64, 64, 128))),
    (512, (0.6, 0.8), (64, 128), ((259, 128, 128, 256), (259, 128, 128, 256))),
]


def _fold(p):
    s = p["g"] * jax.lax.rsqrt(p["v"] + 1e-5)
    w = p["W"] * s[:, None]
    b = (p["b"] - p["m"]) * s + p["be"]
    return w.T, b[None, :]


def _fps_body(x_ref, o_ref, *, P, N8):
    x = x_ref[0]
    niota = (jax.lax.broadcasted_iota(jnp.int32, (8, N8), 0) * N8
             + jax.lax.broadcasted_iota(jnp.int32, (8, N8), 1))
    piota = jax.lax.broadcasted_iota(jnp.int32, (1, P), 1)
    col0 = x[:, 0:1, 0:1]
    acc0 = jnp.where(piota == 0, col0.reshape(3, 1), 0.0)

    def step(i, carry):
        dists, last, acc = carry
        diff = x - last
        d = (diff[0] * diff[0] + diff[1] * diff[1]) + diff[2] * diff[2]
        dists = jnp.minimum(dists, d)
        m = jnp.max(dists)
        nxt = jnp.min(jnp.where(dists == m, niota, jnp.int32(8 * N8)))
        sel = (niota == nxt)[None]
        col = jnp.sum(jnp.where(sel, x, 0.0), axis=(1, 2), keepdims=True)
        acc = jnp.where(piota == i, col.reshape(3, 1), acc)
        return dists, col, acc

    dists0 = jnp.full((8, N8), 1e10, jnp.float32)
    _, _, acc = jax.lax.fori_loop(1, P, step, (dists0, col0, acc0))
    o_ref[0] = acc


def _fps_t(xyzt, npoint):
    b, _, n = xyzt.shape
    n8 = n // 8
    xr = xyzt.reshape(b, 3, 8, n8)
    return pl.pallas_call(
        functools.partial(_fps_body, P=npoint, N8=n8),
        grid=(b,),
        in_specs=[pl.BlockSpec((1, 3, 8, n8), lambda i: (i, 0, 0, 0))],
        out_specs=pl.BlockSpec((1, 3, npoint), lambda i: (i, 0, 0)),
        out_shape=jax.ShapeDtypeStruct((b, 3, npoint), jnp.float32),
    )(xr)


def _bq_body(c_ref, xt_ref, wx1_ref, wx2_ref, idx1_ref, v1_ref, q1_ref,
             idx2_ref, v2_ref, q2_ref, *, N, r1, k1, r2, k2):
    bi = pl.program_id(0)
    c = c_ref[0]
    x = xt_ref[0]
    dx = c[:, 0:1] - x[0:1, :]
    dy = c[:, 1:2] - x[1:2, :]
    dz = c[:, 2:3] - x[2:3, :]
    d = (dx * dx + dy * dy) + dz * dz
    cb = c.shape[0]
    niota = jax.lax.broadcasted_iota(jnp.int32, (cb, N), 1)
    off = bi * N

    for (r, k, idx_ref, v_ref, q_ref, wx_ref) in (
            (r1, k1, idx1_ref, v1_ref, q1_ref, wx1_ref),
            (r2, k2, idx2_ref, v2_ref, q2_ref, wx2_ref)):
        keys = jnp.where(d < r * r, niota, jnp.int32(N))
        for j in range(k):
            mn = jnp.min(keys, axis=1, keepdims=True)
            valid = mn < N
            idx_ref[:, j:j + 1] = jnp.where(valid, mn + off, off)
            v_ref[:, j:j + 1] = valid.astype(jnp.int32)
            if j + 1 < k:
                keys = jnp.where(keys == mn, jnp.int32(N), keys)
        q_ref[...] = jnp.dot(c, wx_ref[...],
                             preferred_element_type=jnp.float32)


def _ball_query(new_c, xyzt, wx1, wx2, scale_cfg, cb=256):
    (r1, k1), (r2, k2) = scale_cfg
    b, p, _ = new_c.shape
    n = xyzt.shape[2]
    cb = min(cb, p)
    nb = p // cb
    c1a, c1b = wx1.shape[1], wx2.shape[1]
    grid = (b, nb)
    outs = pl.pallas_call(
        functools.partial(_bq_body, N=n, r1=r1, k1=k1, r2=r2, k2=k2),
        grid=grid,
        in_specs=[
            pl.BlockSpec((1, cb, 3), lambda i, j: (i, j, 0)),
            pl.BlockSpec((1, 3, n), lambda i, j: (i, 0, 0)),
            pl.BlockSpec((3, c1a), lambda i, j: (0, 0)),
            pl.BlockSpec((3, c1b), lambda i, j: (0, 0)),
        ],
        out_specs=[
            pl.BlockSpec((cb, k1), lambda i, j, _nb=nb: (i * _nb + j, 0)),
            pl.BlockSpec((cb, k1), lambda i, j, _nb=nb: (i * _nb + j, 0)),
            pl.BlockSpec((cb, c1a), lambda i, j, _nb=nb: (i * _nb + j, 0)),
            pl.BlockSpec((cb, k2), lambda i, j, _nb=nb: (i * _nb + j, 0)),
            pl.BlockSpec((cb, k2), lambda i, j, _nb=nb: (i * _nb + j, 0)),
            pl.BlockSpec((cb, c1b), lambda i, j, _nb=nb: (i * _nb + j, 0)),
        ],
        out_shape=[
            jax.ShapeDtypeStruct((b * p, k1), jnp.int32),
            jax.ShapeDtypeStruct((b * p, k1), jnp.int32),
            jax.ShapeDtypeStruct((b * p, c1a), jnp.float32),
            jax.ShapeDtypeStruct((b * p, k2), jnp.int32),
            jax.ShapeDtypeStruct((b * p, k2), jnp.int32),
            jax.ShapeDtypeStruct((b * p, c1b), jnp.float32),
        ],
    )(new_c, xyzt, wx1, wx2)
    return outs


def _proj_body(xt_ref, *refs, has_feats):
    if has_feats:
        f_ref = refs[0]
        refs = refs[1:]
    wf1_ref, b1_ref, wf2_ref, b2_ref, p1_ref, p2_ref = refs[-6:]
    wx1_ref, wx2_ref = refs[0], refs[1]
    xt = xt_ref[0]
    for (wx_ref, wf_ref, b_ref, p_ref) in ((wx1_ref, wf1_ref, b1_ref, p1_ref),
                                           (wx2_ref, wf2_ref, b2_ref, p2_ref)):
        p = jax.lax.dot_general(xt, wx_ref[...], (((0,), (0,)), ((), ())),
                                preferred_element_type=jnp.float32)
        if has_feats:
            p = p + jnp.dot(f_ref[0], wf_ref[...],
                            preferred_element_type=jnp.float32)
        p_ref[...] = p + b_ref[...]


def _project(xyzt, feats, wx1, wf1, b1, wx2, wf2, b2):
    b, _, n = xyzt.shape
    c1a, c1b = wx1.shape[1], wx2.shape[1]
    has_feats = feats is not None
    in_specs = [pl.BlockSpec((1, 3, n), lambda i: (i, 0, 0))]
    args = [xyzt]
    if has_feats:
        cf = feats.shape[2]
        in_specs.append(pl.BlockSpec((1, n, cf), lambda i: (i, 0, 0)))
        args.append(feats)
    for w in (wx1, wx2):
        in_specs.append(pl.BlockSpec(w.shape, lambda i: (0, 0)))
        args.append(w)
    if has_feats:
        wfs = [wf1, b1, wf2, b2]
    else:
        wfs = [jnp.zeros((1, 1), jnp.float32), b1,
               jnp.zeros((1, 1), jnp.float32), b2]
    for w in wfs:
        in_specs.append(pl.BlockSpec(w.shape, lambda i: (0, 0)))
        args.append(w)

    def body(xt_ref, *refs):
        if has_feats:
            _proj_body(xt_ref, refs[0], refs[1], refs[2], *refs[3:],
                       has_feats=True)
        else:
            _proj_body(xt_ref, refs[0], refs[1], *refs[2:], has_feats=False)

    return pl.pallas_call(
        body,
        grid=(b,),
        in_specs=in_specs,
        out_specs=[pl.BlockSpec((n, c1a), lambda i: (i, 0)),
                   pl.BlockSpec((n, c1b), lambda i: (i, 0))],
        out_shape=[jax.ShapeDtypeStruct((b * n, c1a), jnp.float32),
                   jax.ShapeDtypeStruct((b * n, c1b), jnp.float32)],
    )(*args)


def _sc_gather(table, idx):
    r, d = table.shape
    bsz = idx.shape[0]
    nw = 32
    rows_w = bsz // nw
    ch = 128
    nch = rows_w // ch
    assert rows_w % ch == 0, (bsz, rows_w)
    mesh = plsc.VectorSubcoreMesh(core_axis_name="c", subcore_axis_name="s")

    @functools.partial(
        pl.kernel,
        out_type=jax.ShapeDtypeStruct((bsz, d), jnp.float32),
        mesh=mesh,
        compiler_params=pltpu.CompilerParams(use_tc_tiling_on_sc=False),
        scratch_types=[
            pltpu.VMEM((ch,), jnp.int32),
            pltpu.VMEM((ch, d), jnp.float32),
            pltpu.SemaphoreType.DMA,
        ],
    )
    def k(table_hbm, idx_hbm, out_hbm, idx_v, rows_v, sem):
        wid = jax.lax.axis_index("s") * 2 + jax.lax.axis_index("c")
        base = wid * rows_w

        def body(j, _):
            off = base + j * ch
            pltpu.sync_copy(idx_hbm.at[pl.ds(off, ch)], idx_v)
            pltpu.async_copy(table_hbm.at[idx_v], rows_v, sem).wait()
            pltpu.sync_copy(rows_v, out_hbm.at[pl.ds(off, ch)])
            return 0

        jax.lax.fori_loop(0, nch, body, 0, unroll=False)

    return k(table, idx)


def _mlp_body(g_ref, q_ref, v_ref, w2_ref, b2_ref, w3_ref, b3_ref, o_ref, *,
              M, K):
    q = q_ref[...]
    c1 = q.shape[1]
    g = g_ref[...].reshape(M, K, c1)
    h1 = jnp.maximum(g - q[:, None, :], 0.0).reshape(M * K, c1)
    h2 = jnp.maximum(jnp.dot(h1, w2_ref[...],
                             preferred_element_type=jnp.float32) + b2_ref[...],
                     0.0)
    h3 = jnp.maximum(jnp.dot(h2, w3_ref[...],
                             preferred_element_type=jnp.float32) + b3_ref[...],
                     0.0)
    c3 = h3.shape[1]
    h3 = h3.reshape(M, K, c3)
    v = v_ref[...][:, :, None] > 0
    o_ref[...] = jnp.max(jnp.where(v, h3, 0.0), axis=1)


def _mlp_pool(g, q, valid, w2, b2, w3, b3, k):
    bp = q.shape[0]
    c1 = q.shape[1]
    c2, c3 = w2.shape[1], w3.shape[1]
    m = max(8, min(128, 2048 // k))
    while bp % m:
        m //= 2
    grid = (bp // m,)
    return pl.pallas_call(
        functools.partial(_mlp_body, M=m, K=k),
        grid=grid,
        in_specs=[
            pl.BlockSpec((m * k, c1), lambda j: (j, 0)),
            pl.BlockSpec((m, c1), lambda j: (j, 0)),
            pl.BlockSpec((m, k), lambda j: (j, 0)),
            pl.BlockSpec((c1, c2), lambda j: (0, 0)),
            pl.BlockSpec((1, c2), lambda j: (0, 0)),
            pl.BlockSpec((c2, c3), lambda j: (0, 0)),
            pl.BlockSpec((1, c3), lambda j: (0, 0)),
        ],
        out_specs=pl.BlockSpec((m, c3), lambda j: (j, 0)),
        out_shape=jax.ShapeDtypeStruct((bp, c3), jnp.float32),
    )(g, q, valid, w2, b2, w3, b3)


def _head_body(ctx_ref, xyzt_ref, *refs):
    wb, outs = refs[:10], refs[10:]
    skel_ref, r_ref, cmb_ref = outs
    h = ctx_ref[...]
    for i in range(5):
        w, b = wb[2 * i][...], wb[2 * i + 1][...]
        h = jnp.dot(h, w, preferred_element_type=jnp.float32) + b
        if i < 4:
            h = jnp.maximum(h, 0.0)
    m = jnp.max(h, axis=0, keepdims=True)
    e = jnp.exp(h - m)
    sm = e / jnp.sum(e, axis=0, keepdims=True)
    xyzt = xyzt_ref[...]
    skel = jax.lax.dot_general(sm, xyzt, (((0,), (1,)), ((), ())),
                               preferred_element_type=jnp.float32)
    cmb = jax.lax.dot_general(sm, ctx_ref[...], (((0,), (0,)), ((), ())),
                              preferred_element_type=jnp.float32)
    dx = skel[:, 0:1] - xyzt[0:1, :]
    dy = skel[:, 1:2] - xyzt[1:2, :]
    dz = skel[:, 2:3] - xyzt[2:3, :]
    dt = jnp.sqrt(dx * dx + dy * dy + dz * dz + 1e-12)
    mind = jnp.min(dt, axis=0, keepdims=True)
    skel_r = jax.lax.dot_general(sm, mind, (((0,), (1,)), ((), ())),
                                 preferred_element_type=jnp.float32)
    skel_ref[...] = skel
    r_ref[...] = skel_r
    cmb_ref[...] = cmb


def _head(context, xyzt, head_params):
    b, s, c = context.shape
    k = head_params[-1]["W"].shape[0]
    wbs = []
    for p in head_params:
        w, bb = _fold(p)
        wbs += [w, bb]
    grid_spec = pl.GridSpec(
        grid=(b,),
        in_specs=[
            pl.BlockSpec((1, s, c), lambda i: (i, 0, 0)),
            pl.BlockSpec((1, 3, s), lambda i: (i, 0, 0)),
        ] + [pl.BlockSpec(w.shape, lambda i: (0,) * w.ndim) for w in wbs],
        out_specs=[
            pl.BlockSpec((1, k, 3), lambda i: (i, 0, 0)),
            pl.BlockSpec((1, k, 1), lambda i: (i, 0, 0)),
            pl.BlockSpec((1, k, c), lambda i: (i, 0, 0)),
        ],
    )

    def body(ctx_ref, xyzt_ref, *refs):
        _head_body(ctx_ref.at[0], xyzt_ref.at[0], *refs[:10],
                   refs[10].at[0], refs[11].at[0], refs[12].at[0])

    out = pl.pallas_call(
        body,
        grid_spec=grid_spec,
        out_shape=[
            jax.ShapeDtypeStruct((b, k, 3), jnp.float32),
            jax.ShapeDtypeStruct((b, k, 1), jnp.float32),
            jax.ShapeDtypeStruct((b, k, c), jnp.float32),
        ],
    )(context, xyzt, *wbs)
    return out


def _sa_stage(xyzt, feats, cfg, scale_params):
    npoint, radii, nsamples, _ = cfg
    b, _, n = xyzt.shape
    newt = _fps_t(xyzt, npoint)
    new_c = jnp.transpose(newt, (0, 2, 1))

    folded = []
    for layers in scale_params:
        folded.append([_fold(lp) for lp in layers])
    (w1a, b1a), (w2a, b2a), (w3a, b3a) = folded[0]
    (w1b, b1b), (w2b, b2b), (w3b, b3b) = folded[1]
    wx1, wf1 = w1a[:3], w1a[3:]
    wx2, wf2 = w1b[:3], w1b[3:]

    p1, p2 = _project(xyzt, feats, wx1, wf1, b1a, wx2, wf2, b1b)
    idx1, v1, q1, idx2, v2, q2 = _ball_query(
        new_c, xyzt, wx1, wx2, tuple(zip(radii, nsamples)))

    outs = []
    for (idx, v, q, w2, bb2, w3, bb3, p, k) in (
            (idx1, v1, q1, w2a, b2a, w3a, b3a, p1, nsamples[0]),
            (idx2, v2, q2, w2b, b2b, w3b, b3b, p2, nsamples[1])):
        g = _sc_gather(p, idx.reshape(-1))
        outs.append(_mlp_pool(g, q, v, w2, bb2, w3, bb3, k))
    feats_out = jnp.concatenate(outs, axis=-1).reshape(b, npoint, -1)
    return newt, feats_out


def kernel(input_pc, params):
    xyzt = jnp.transpose(input_pc[..., 0:3], (0, 2, 1))
    feats = None
    for cfg, sp in zip(_SA_CFG, params["sa"]):
        xyzt, feats = _sa_stage(xyzt, feats, cfg, sp)
    skel_xyz, skel_r, shape_cmb = _head(feats, xyzt, params["head"])
    return skel_xyz, skel_r, shape_cmb


# speedup vs baseline: 11.9691x; 1.6010x over previous
import functools

import jax
import jax.numpy as jnp
from jax.experimental import pallas as pl
from jax.experimental.pallas import tpu as pltpu
from jax.experimental.pallas import tpu_sc as plsc

_SA_CFG = [
    (1024, (0.1, 0.2), (16, 32), ((3, 16, 16, 32), (3, 16, 16, 32))),
    (768, (0.2, 0.4), (32, 64), ((67, 32, 32, 64), (67, 32, 32, 64))),
    (512, (0.4, 0.6), (32, 64), ((131, 64, 64, 128), (131, 64, 64, 128))),
    (512, (0.6, 0.8), (64, 128), ((259, 128, 128, 256), (259, 128, 128, 256))),
]


def _fold(p):
    s = p["g"] * jax.lax.rsqrt(p["v"] + 1e-5)
    w = p["W"] * s[:, None]
    b = (p["b"] - p["m"]) * s + p["be"]
    return w.T, b[None, :]


def _fps_body(x_ref, o_ref, *, B, P, N8):
    x = x_ref[...]
    niota = (jax.lax.broadcasted_iota(jnp.int32, (1, 8, N8), 1) * N8
             + jax.lax.broadcasted_iota(jnp.int32, (1, 8, N8), 2))
    piota = jax.lax.broadcasted_iota(jnp.int32, (1, 1, P), 2)
    col0 = x[:, :, 0:1, 0:1]
    acc0 = jnp.where(piota == 0, col0.reshape(B, 3, 1), 0.0)

    def step(i, carry):
        dists, last, acc = carry
        diff = x - last
        d = ((diff[:, 0] * diff[:, 0] + diff[:, 1] * diff[:, 1])
             + diff[:, 2] * diff[:, 2])
        dists = jnp.minimum(dists, d)
        m = jnp.max(dists, axis=(1, 2), keepdims=True)
        nxt = jnp.min(jnp.where(dists == m, niota, jnp.int32(8 * N8)),
                      axis=(1, 2), keepdims=True)
        sel = (niota == nxt)[:, None]
        col = jnp.sum(jnp.where(sel, x, 0.0), axis=(2, 3),
                      keepdims=True)
        acc = jnp.where(piota == i, col.reshape(B, 3, 1), acc)
        return dists, col, acc

    dists0 = jnp.full((B, 8, N8), 1e10, jnp.float32)
    _, _, acc = jax.lax.fori_loop(1, P, step, (dists0, col0, acc0))
    o_ref[...] = acc


def _fps_t(xyzt, npoint):
    b, _, n = xyzt.shape
    n8 = n // 8
    xr = xyzt.reshape(b, 3, 8, n8)
    return pl.pallas_call(
        functools.partial(_fps_body, B=b, P=npoint, N8=n8),
        in_specs=[pl.BlockSpec((b, 3, 8, n8), lambda: (0, 0, 0, 0))],
        out_specs=pl.BlockSpec((b, 3, npoint), lambda: (0, 0, 0)),
        out_shape=jax.ShapeDtypeStruct((b, 3, npoint), jnp.float32),
    )(xr)


def _bq_body(c_ref, xt_ref, wx1_ref, wx2_ref, idx1_ref, v1_ref, q1_ref,
             idx2_ref, v2_ref, q2_ref, *, N, r1, k1, r2, k2):
    bi = pl.program_id(0)
    c = c_ref[0]
    x = xt_ref[0]
    dx = c[:, 0:1] - x[0:1, :]
    dy = c[:, 1:2] - x[1:2, :]
    dz = c[:, 2:3] - x[2:3, :]
    d = (dx * dx + dy * dy) + dz * dz
    cb = c.shape[0]
    niota = jax.lax.broadcasted_iota(jnp.int32, (cb, N), 1)
    off = bi * N

    for (r, k, idx_ref, v_ref, q_ref, wx_ref) in (
            (r1, k1, idx1_ref, v1_ref, q1_ref, wx1_ref),
            (r2, k2, idx2_ref, v2_ref, q2_ref, wx2_ref)):
        keys = jnp.where(d < r * r, niota, jnp.int32(N))
        for j in range(k):
            mn = jnp.min(keys, axis=1, keepdims=True)
            valid = mn < N
            idx_ref[:, j:j + 1] = jnp.where(valid, mn + off, off)
            v_ref[:, j:j + 1] = valid.astype(jnp.int32)
            if j + 1 < k:
                keys = jnp.where(keys == mn, jnp.int32(N), keys)
        q_ref[...] = jnp.dot(c, wx_ref[...],
                             preferred_element_type=jnp.float32)


def _ball_query(new_c, xyzt, wx1, wx2, scale_cfg, cb=256):
    (r1, k1), (r2, k2) = scale_cfg
    b, p, _ = new_c.shape
    n = xyzt.shape[2]
    cb = min(cb, p)
    nb = p // cb
    c1a, c1b = wx1.shape[1], wx2.shape[1]
    grid = (b, nb)
    outs = pl.pallas_call(
        functools.partial(_bq_body, N=n, r1=r1, k1=k1, r2=r2, k2=k2),
        grid=grid,
        in_specs=[
            pl.BlockSpec((1, cb, 3), lambda i, j: (i, j, 0)),
            pl.BlockSpec((1, 3, n), lambda i, j: (i, 0, 0)),
            pl.BlockSpec((3, c1a), lambda i, j: (0, 0)),
            pl.BlockSpec((3, c1b), lambda i, j: (0, 0)),
        ],
        out_specs=[
            pl.BlockSpec((cb, k1), lambda i, j, _nb=nb: (i * _nb + j, 0)),
            pl.BlockSpec((cb, k1), lambda i, j, _nb=nb: (i * _nb + j, 0)),
            pl.BlockSpec((cb, c1a), lambda i, j, _nb=nb: (i * _nb + j, 0)),
            pl.BlockSpec((cb, k2), lambda i, j, _nb=nb: (i * _nb + j, 0)),
            pl.BlockSpec((cb, k2), lambda i, j, _nb=nb: (i * _nb + j, 0)),
            pl.BlockSpec((cb, c1b), lambda i, j, _nb=nb: (i * _nb + j, 0)),
        ],
        out_shape=[
            jax.ShapeDtypeStruct((b * p, k1), jnp.int32),
            jax.ShapeDtypeStruct((b * p, k1), jnp.int32),
            jax.ShapeDtypeStruct((b * p, c1a), jnp.float32),
            jax.ShapeDtypeStruct((b * p, k2), jnp.int32),
            jax.ShapeDtypeStruct((b * p, k2), jnp.int32),
            jax.ShapeDtypeStruct((b * p, c1b), jnp.float32),
        ],
    )(new_c, xyzt, wx1, wx2)
    return outs


def _proj_body(xt_ref, *refs, has_feats):
    if has_feats:
        f_ref = refs[0]
        refs = refs[1:]
    wf1_ref, b1_ref, wf2_ref, b2_ref, p1_ref, p2_ref = refs[-6:]
    wx1_ref, wx2_ref = refs[0], refs[1]
    xt = xt_ref[0]
    for (wx_ref, wf_ref, b_ref, p_ref) in ((wx1_ref, wf1_ref, b1_ref, p1_ref),
                                           (wx2_ref, wf2_ref, b2_ref, p2_ref)):
        p = jax.lax.dot_general(xt, wx_ref[...], (((0,), (0,)), ((), ())),
                                preferred_element_type=jnp.float32)
        if has_feats:
            p = p + jnp.dot(f_ref[0], wf_ref[...],
                            preferred_element_type=jnp.float32)
        p_ref[...] = p + b_ref[...]


def _project(xyzt, feats, wx1, wf1, b1, wx2, wf2, b2):
    b, _, n = xyzt.shape
    c1a, c1b = wx1.shape[1], wx2.shape[1]
    has_feats = feats is not None
    in_specs = [pl.BlockSpec((1, 3, n), lambda i: (i, 0, 0))]
    args = [xyzt]
    if has_feats:
        cf = feats.shape[2]
        in_specs.append(pl.BlockSpec((1, n, cf), lambda i: (i, 0, 0)))
        args.append(feats)
    for w in (wx1, wx2):
        in_specs.append(pl.BlockSpec(w.shape, lambda i: (0, 0)))
        args.append(w)
    if has_feats:
        wfs = [wf1, b1, wf2, b2]
    else:
        wfs = [jnp.zeros((1, 1), jnp.float32), b1,
               jnp.zeros((1, 1), jnp.float32), b2]
    for w in wfs:
        in_specs.append(pl.BlockSpec(w.shape, lambda i: (0, 0)))
        args.append(w)

    def body(xt_ref, *refs):
        if has_feats:
            _proj_body(xt_ref, refs[0], refs[1], refs[2], *refs[3:],
                       has_feats=True)
        else:
            _proj_body(xt_ref, refs[0], refs[1], *refs[2:], has_feats=False)

    return pl.pallas_call(
        body,
        grid=(b,),
        in_specs=in_specs,
        out_specs=[pl.BlockSpec((n, c1a), lambda i: (i, 0)),
                   pl.BlockSpec((n, c1b), lambda i: (i, 0))],
        out_shape=[jax.ShapeDtypeStruct((b * n, c1a), jnp.float32),
                   jax.ShapeDtypeStruct((b * n, c1b), jnp.float32)],
    )(*args)


def _sc_gather(table, idx):
    r, d = table.shape
    bsz = idx.shape[0]
    nw = 32
    rows_w = bsz // nw
    ch = 128
    nch = rows_w // ch
    assert rows_w % ch == 0, (bsz, rows_w)
    mesh = plsc.VectorSubcoreMesh(core_axis_name="c", subcore_axis_name="s")

    @functools.partial(
        pl.kernel,
        out_type=jax.ShapeDtypeStruct((bsz, d), jnp.float32),
        mesh=mesh,
        compiler_params=pltpu.CompilerParams(use_tc_tiling_on_sc=False),
        scratch_types=[
            pltpu.VMEM((ch,), jnp.int32),
            pltpu.VMEM((ch, d), jnp.float32),
            pltpu.SemaphoreType.DMA,
        ],
    )
    def k(table_hbm, idx_hbm, out_hbm, idx_v, rows_v, sem):
        wid = jax.lax.axis_index("s") * 2 + jax.lax.axis_index("c")
        base = wid * rows_w

        def body(j, _):
            off = base + j * ch
            pltpu.sync_copy(idx_hbm.at[pl.ds(off, ch)], idx_v)
            pltpu.async_copy(table_hbm.at[idx_v], rows_v, sem).wait()
            pltpu.sync_copy(rows_v, out_hbm.at[pl.ds(off, ch)])
            return 0

        jax.lax.fori_loop(0, nch, body, 0, unroll=False)

    return k(table, idx)


def _mlp_body(g_ref, q_ref, v_ref, w2_ref, b2_ref, w3_ref, b3_ref, o_ref, *,
              M, K):
    q = q_ref[...]
    c1 = q.shape[1]
    g = g_ref[...].reshape(M, K, c1)
    h1 = jnp.maximum(g - q[:, None, :], 0.0).reshape(M * K, c1)
    h2 = jnp.maximum(jnp.dot(h1, w2_ref[...],
                             preferred_element_type=jnp.float32) + b2_ref[...],
                     0.0)
    h3 = jnp.maximum(jnp.dot(h2, w3_ref[...],
                             preferred_element_type=jnp.float32) + b3_ref[...],
                     0.0)
    c3 = h3.shape[1]
    h3 = h3.reshape(M, K, c3)
    v = v_ref[...][:, :, None] > 0
    o_ref[...] = jnp.max(jnp.where(v, h3, 0.0), axis=1)


def _mlp_pool(g, q, valid, w2, b2, w3, b3, k):
    bp = q.shape[0]
    c1 = q.shape[1]
    c2, c3 = w2.shape[1], w3.shape[1]
    m = max(8, min(128, 2048 // k))
    while bp % m:
        m //= 2
    grid = (bp // m,)
    return pl.pallas_call(
        functools.partial(_mlp_body, M=m, K=k),
        grid=grid,
        in_specs=[
            pl.BlockSpec((m * k, c1), lambda j: (j, 0)),
            pl.BlockSpec((m, c1), lambda j: (j, 0)),
            pl.BlockSpec((m, k), lambda j: (j, 0)),
            pl.BlockSpec((c1, c2), lambda j: (0, 0)),
            pl.BlockSpec((1, c2), lambda j: (0, 0)),
            pl.BlockSpec((c2, c3), lambda j: (0, 0)),
            pl.BlockSpec((1, c3), lambda j: (0, 0)),
        ],
        out_specs=pl.BlockSpec((m, c3), lambda j: (j, 0)),
        out_shape=jax.ShapeDtypeStruct((bp, c3), jnp.float32),
    )(g, q, valid, w2, b2, w3, b3)


def _head_body(ctx_ref, xyzt_ref, *refs):
    wb, outs = refs[:10], refs[10:]
    skel_ref, r_ref, cmb_ref = outs
    h = ctx_ref[...]
    for i in range(5):
        w, b = wb[2 * i][...], wb[2 * i + 1][...]
        h = jnp.dot(h, w, preferred_element_type=jnp.float32) + b
        if i < 4:
            h = jnp.maximum(h, 0.0)
    m = jnp.max(h, axis=0, keepdims=True)
    e = jnp.exp(h - m)
    sm = e / jnp.sum(e, axis=0, keepdims=True)
    xyzt = xyzt_ref[...]
    skel = jax.lax.dot_general(sm, xyzt, (((0,), (1,)), ((), ())),
                               preferred_element_type=jnp.float32)
    cmb = jax.lax.dot_general(sm, ctx_ref[...], (((0,), (0,)), ((), ())),
                              preferred_element_type=jnp.float32)
    dx = skel[:, 0:1] - xyzt[0:1, :]
    dy = skel[:, 1:2] - xyzt[1:2, :]
    dz = skel[:, 2:3] - xyzt[2:3, :]
    dt = jnp.sqrt(dx * dx + dy * dy + dz * dz + 1e-12)
    mind = jnp.min(dt, axis=0, keepdims=True)
    skel_r = jax.lax.dot_general(sm, mind, (((0,), (1,)), ((), ())),
                                 preferred_element_type=jnp.float32)
    skel_ref[...] = skel
    r_ref[...] = skel_r
    cmb_ref[...] = cmb


def _head(context, xyzt, head_params):
    b, s, c = context.shape
    k = head_params[-1]["W"].shape[0]
    wbs = []
    for p in head_params:
        w, bb = _fold(p)
        wbs += [w, bb]
    grid_spec = pl.GridSpec(
        grid=(b,),
        in_specs=[
            pl.BlockSpec((1, s, c), lambda i: (i, 0, 0)),
            pl.BlockSpec((1, 3, s), lambda i: (i, 0, 0)),
        ] + [pl.BlockSpec(w.shape, lambda i: (0,) * w.ndim) for w in wbs],
        out_specs=[
            pl.BlockSpec((1, k, 3), lambda i: (i, 0, 0)),
            pl.BlockSpec((1, k, 1), lambda i: (i, 0, 0)),
            pl.BlockSpec((1, k, c), lambda i: (i, 0, 0)),
        ],
    )

    def body(ctx_ref, xyzt_ref, *refs):
        _head_body(ctx_ref.at[0], xyzt_ref.at[0], *refs[:10],
                   refs[10].at[0], refs[11].at[0], refs[12].at[0])

    out = pl.pallas_call(
        body,
        grid_spec=grid_spec,
        out_shape=[
            jax.ShapeDtypeStruct((b, k, 3), jnp.float32),
            jax.ShapeDtypeStruct((b, k, 1), jnp.float32),
            jax.ShapeDtypeStruct((b, k, c), jnp.float32),
        ],
    )(context, xyzt, *wbs)
    return out


def _sa_stage(xyzt, feats, cfg, scale_params):
    npoint, radii, nsamples, _ = cfg
    b, _, n = xyzt.shape
    newt = _fps_t(xyzt, npoint)
    new_c = jnp.transpose(newt, (0, 2, 1))

    folded = []
    for layers in scale_params:
        folded.append([_fold(lp) for lp in layers])
    (w1a, b1a), (w2a, b2a), (w3a, b3a) = folded[0]
    (w1b, b1b), (w2b, b2b), (w3b, b3b) = folded[1]
    wx1, wf1 = w1a[:3], w1a[3:]
    wx2, wf2 = w1b[:3], w1b[3:]

    p1, p2 = _project(xyzt, feats, wx1, wf1, b1a, wx2, wf2, b1b)
    idx1, v1, q1, idx2, v2, q2 = _ball_query(
        new_c, xyzt, wx1, wx2, tuple(zip(radii, nsamples)))

    outs = []
    for (idx, v, q, w2, bb2, w3, bb3, p, k) in (
            (idx1, v1, q1, w2a, b2a, w3a, b3a, p1, nsamples[0]),
            (idx2, v2, q2, w2b, b2b, w3b, b3b, p2, nsamples[1])):
        g = _sc_gather(p, idx.reshape(-1))
        outs.append(_mlp_pool(g, q, v, w2, bb2, w3, bb3, k))
    feats_out = jnp.concatenate(outs, axis=-1).reshape(b, npoint, -1)
    return newt, feats_out


def kernel(input_pc, params):
    xyzt = jnp.transpose(input_pc[..., 0:3], (0, 2, 1))
    feats = None
    for cfg, sp in zip(_SA_CFG, params["sa"]):
        xyzt, feats = _sa_stage(xyzt, feats, cfg, sp)
    skel_xyz, skel_r, shape_cmb = _head(feats, xyzt, params["head"])
    return skel_xyz, skel_r, shape_cmb
